# baseline (speedup 1.0000x reference)
import functools
import math

import jax
import jax.numpy as jnp
from jax.experimental import pallas as pl
from jax.experimental.pallas import tpu as pltpu

NS, NG, DC, DG, H, GDF, K = 10000, 2500, 128, 16, 256, 16, 16
E = NS * K
BNS = 1.0 / math.sqrt(1.0 + 1e-5)


def _bn(x, g, b):
    return x * (BNS * g) + b


def _silu(x):
    return x * jax.nn.sigmoid(x)


def _chem_body(chem_ref, cw1_ref, cb1_ref, cg1_ref, cbe1_ref, cw2_ref, cb2_ref,
               cg2_ref, cbe2_ref, sw1c_ref, sb1_ref, hchem_ref, proj_ref):
    x = chem_ref[...]
    h = _bn(jnp.dot(x, cw1_ref[...], preferred_element_type=jnp.float32)
            + cb1_ref[...], cg1_ref[...], cbe1_ref[...])
    h = _silu(h)
    h = _bn(jnp.dot(h, cw2_ref[...], preferred_element_type=jnp.float32)
            + cb2_ref[...], cg2_ref[...], cbe2_ref[...])
    hchem_ref[...] = h
    proj_ref[...] = jnp.dot(x, sw1c_ref[...], preferred_element_type=jnp.float32)


def _chem_prep(chem_feats, cw1, cb1, cg1, cbe1, cw2, cb2, cg2, cbe2, sw1c, sb1):
    return pl.pallas_call(
        _chem_body,
        out_shape=(jax.ShapeDtypeStruct((NG, H), jnp.float32),
                   jax.ShapeDtypeStruct((NG, H), jnp.float32)),
    )(chem_feats, cw1, cb1, cg1, cbe1, cw2, cb2, cg2, cbe2, sw1c, sb1)


def _geom_body(geom_ref, verts_ref, npT_ref, gw1_ref, gb1_ref, gg1_ref,
               gbe1_ref, gw2_ref, gb2_ref, gg2_ref, gbe2_ref,
               hg_ref, d2_ref):
    x = geom_ref[...]
    h = _bn(jnp.dot(x, gw1_ref[...], preferred_element_type=jnp.float32)
            + gb1_ref[...], gg1_ref[...], gbe1_ref[...])
    h = _silu(h)
    h = _bn(jnp.dot(h, gw2_ref[...], preferred_element_type=jnp.float32)
            + gb2_ref[...], gg2_ref[...], gbe2_ref[...])
    hg_ref[...] = h
    v = verts_ref[...]
    npT = npT_ref[...]
    vsq = jnp.sum(v * v, axis=1, keepdims=True)
    nsq = jnp.sum(npT * npT, axis=0, keepdims=True)
    d2_ref[...] = vsq + nsq - 2.0 * jnp.dot(v, npT,
                                            preferred_element_type=jnp.float32)


def _geom_prep(geom_feats, verts, npT, gw1, gb1, gg1, gbe1, gw2, gb2, gg2, gbe2):
    B = 1000
    grid = NS // B
    return pl.pallas_call(
        _geom_body,
        grid=(grid,),
        in_specs=[
            pl.BlockSpec((B, DG), lambda i: (i, 0)),
            pl.BlockSpec((B, 3), lambda i: (i, 0)),
            pl.BlockSpec((3, NG), lambda i: (0, 0)),
            pl.BlockSpec((DG, H), lambda i: (0, 0)),
            pl.BlockSpec((1, H), lambda i: (0, 0)),
            pl.BlockSpec((1, H), lambda i: (0, 0)),
            pl.BlockSpec((1, H), lambda i: (0, 0)),
            pl.BlockSpec((H, H), lambda i: (0, 0)),
            pl.BlockSpec((1, H), lambda i: (0, 0)),
            pl.BlockSpec((1, H), lambda i: (0, 0)),
            pl.BlockSpec((1, H), lambda i: (0, 0)),
        ],
        out_specs=[
            pl.BlockSpec((B, H), lambda i: (i, 0)),
            pl.BlockSpec((B, NG), lambda i: (i, 0)),
        ],
        out_shape=(jax.ShapeDtypeStruct((NS, H), jnp.float32),
                   jax.ShapeDtypeStruct((NS, NG), jnp.float32)),
    )(geom_feats, verts, npT, gw1, gb1, gg1, gbe1, gw2, gb2, gg2, gbe2)


def _topk_body(d2_ref, idx_ref, scratch_ref):
    BT = d2_ref.shape[0]
    scratch_ref[...] = d2_ref[...]
    colid = jax.lax.broadcasted_iota(jnp.int32, (BT, NG), 1)
    kid = jax.lax.broadcasted_iota(jnp.int32, (BT, K), 1)

    def step(i, acc):
        d = scratch_ref[...]
        m = jnp.min(d, axis=1, keepdims=True)
        am = jnp.min(jnp.where(d == m, colid, NG), axis=1)
        scratch_ref[...] = jnp.where(colid == am[:, None], jnp.inf, d)
        return jnp.where(kid == i, am[:, None], acc)

    idx_ref[...] = jax.lax.fori_loop(0, K, step,
                                     jnp.zeros((BT, K), jnp.int32))


def _topk(d2):
    BT = 400
    grid = NS // BT
    return pl.pallas_call(
        _topk_body,
        grid=(grid,),
        in_specs=[pl.BlockSpec((BT, NG), lambda i: (i, 0))],
        out_specs=pl.BlockSpec((BT, K), lambda i: (i, 0)),
        out_shape=jax.ShapeDtypeStruct((NS, K), jnp.int32),
        scratch_shapes=[pltpu.VMEM((BT, NG), jnp.float32)],
    )(d2)


def _edge_body(chem_g_ref, npg_ref, verts_ref, nrm_ref, hg1_ref, mu_ref,
               sw1_ref, sb1_ref, sg1_ref, sbe1_ref, sw2_ref, sb2_ref,
               sg2_ref, sbe2_ref, fw1_ref, fb1_ref, fg1_ref, fbe1_ref,
               fw2_ref, fb2_ref, fg2_ref, fbe2_ref, out_ref):
    EB = npg_ref.shape[0]
    BV = EB // K

    npg = npg_ref[...]
    dx = npg[:, 0:1] - verts_ref[:, 0:1]
    dy = npg[:, 1:2] - verts_ref[:, 1:2]
    dz = npg[:, 2:3] - verts_ref[:, 2:3]
    dist = jnp.sqrt(dx * dx + dy * dy + dz * dz)
    ang = (dx * nrm_ref[:, 0:1] + dy * nrm_ref[:, 1:2]
           + dz * nrm_ref[:, 2:3]) / dist

    mu_d = mu_ref[0:1, :]
    mu_a = mu_ref[1:2, :]
    enc_d = jnp.exp(-(((dist - mu_d) / 0.5) ** 2))
    enc_a = jnp.exp(-(((ang - mu_a) / 0.125) ** 2))

    enc = jnp.concatenate([enc_d, enc_a], axis=1)
    x1 = (jnp.dot(chem_g_ref[...], sw1_ref[:DC, :].astype(jnp.bfloat16),
                  preferred_element_type=jnp.float32)
          + jnp.dot(enc, sw1_ref[DC:, :], preferred_element_type=jnp.float32))
    h = _bn(x1 + sb1_ref[...], sg1_ref[...], sbe1_ref[...])
    h = _silu(h)
    h = _bn(jnp.dot(h.astype(jnp.bfloat16), sw2_ref[...].astype(jnp.bfloat16),
                    preferred_element_type=jnp.float32)
            + sb2_ref[...], sg2_ref[...], sbe2_ref[...])
    filt, core = h[:, :H], h[:, H:]
    he = jax.nn.sigmoid(filt) * jax.nn.softplus(core)
    h_cg = jnp.sum(he.reshape(BV, K, H), axis=1)

    y = jnp.concatenate([h_cg, hg1_ref[...]], axis=1)
    y = _bn(jnp.dot(y, fw1_ref[...], preferred_element_type=jnp.float32)
            + fb1_ref[...], fg1_ref[...], fbe1_ref[...])
    y = _silu(y)
    y = _bn(jnp.dot(y, fw2_ref[...], preferred_element_type=jnp.float32)
            + fb2_ref[...], fg2_ref[...], fbe2_ref[...])
    out_ref[...] = y


def _edge_final(chem_g, npg, verts, nrm, hg1, mu, sw1, sb1, sg1, sbe1, sw2,
                sb2, sg2, sbe2, fw1, fb1, fg1, fbe1, fw2, fb2, fg2, fbe2):
    BV = 400
    EB = BV * K
    grid = NS // BV
    const = lambda shape: pl.BlockSpec(shape, lambda i: (0, 0))
    return pl.pallas_call(
        _edge_body,
        grid=(grid,),
        in_specs=[
            pl.BlockSpec((EB, DC), lambda i: (i, 0)),
            pl.BlockSpec((EB, 16), lambda i: (i, 0)),
            pl.BlockSpec((EB, 3), lambda i: (i, 0)),
            pl.BlockSpec((EB, 3), lambda i: (i, 0)),
            pl.BlockSpec((BV, H), lambda i: (i, 0)),
            const((2, GDF)),
            const((DC + 2 * GDF, H)), const((1, H)), const((1, H)), const((1, H)),
            const((H, 2 * H)), const((1, 2 * H)), const((1, 2 * H)), const((1, 2 * H)),
            const((2 * H, H)), const((1, H)), const((1, H)), const((1, H)),
            const((H, H)), const((1, H)), const((1, H)), const((1, H)),
        ],
        out_specs=pl.BlockSpec((BV, H), lambda i: (i, 0)),
        out_shape=jax.ShapeDtypeStruct((NS, H), jnp.float32),
    )(chem_g, npg, verts, nrm, hg1, mu, sw1, sb1, sg1, sbe1, sw2, sb2, sg2,
      sbe2, fw1, fb1, fg1, fbe1, fw2, fb2, fg2, fbe2)


def kernel(chem_feats, geom_feats, verts, node_pos, cw1, cb1, cg1, cbe1, cw2,
           cb2, cg2, cbe2, gw1, gb1, gg1, gbe1, gw2, gb2, gg2, gbe2, sw1, sb1,
           sg1, sbe1, sw2, sb2, sg2, sbe2, fw1, fb1, fg1, fbe1, fw2, fb2, fg2,
           fbe2):
    row = lambda b: b.reshape(1, -1)
    h_chem, _ = _chem_prep(chem_feats, cw1, row(cb1), row(cg1), row(cbe1),
                           cw2, row(cb2), row(cg2), row(cbe2),
                           sw1[:DC], row(sb1))
    npT = node_pos.T
    hg1, d2 = _geom_prep(geom_feats, verts, npT, gw1, row(gb1), row(gg1),
                         row(gbe1), gw2, row(gb2), row(gg2), row(gbe2))
    idx = _topk(d2)
    flat = idx.reshape(-1)
    chem_g = chem_feats.astype(jnp.bfloat16)[flat]
    np16 = jnp.pad(node_pos, ((0, 0), (0, 13)))
    npg = np16[flat]
    verts_e = jnp.repeat(verts, K, axis=0)
    nrm_e = jnp.repeat(geom_feats[:, DG - 3:], K, axis=0)
    mu = jnp.stack([jnp.linspace(0.0, 8.0, GDF),
                    jnp.linspace(-1.0, 1.0, GDF)]).astype(jnp.float32)
    out = _edge_final(chem_g, npg, verts_e, nrm_e, hg1, mu, sw1, row(sb1), row(sg1),
                      row(sbe1), sw2, row(sb2), row(sg2), row(sbe2), fw1,
                      row(fb1), row(fg1), row(fbe1), fw2, row(fb2), row(fg2),
                      row(fbe2))
    return (out, h_chem)


# speedup vs baseline: 1.0481x; 1.0481x over previous
import functools
import math

import jax
import jax.numpy as jnp
from jax import lax
from jax.experimental import pallas as pl
from jax.experimental.pallas import tpu as pltpu
from jax.experimental.pallas import tpu_sc as plsc

NS, NG, DC, DG, H, GDF, K = 10000, 2500, 128, 16, 256, 16, 16
E = NS * K
BNS = 1.0 / math.sqrt(1.0 + 1e-5)

_SC_CORES, _SC_SUBCORES = 2, 16
_NW = _SC_CORES * _SC_SUBCORES


def _bn(x, g, b):
    return x * (BNS * g) + b


def _silu(x):
    return x * jax.nn.sigmoid(x)


def _chem_body(chem_ref, cw1_ref, cb1_ref, cg1_ref, cbe1_ref, cw2_ref, cb2_ref,
               cg2_ref, cbe2_ref, sw1c_ref, sb1_ref, hchem_ref, proj_ref):
    x = chem_ref[...]
    h = _bn(jnp.dot(x, cw1_ref[...], preferred_element_type=jnp.float32)
            + cb1_ref[...], cg1_ref[...], cbe1_ref[...])
    h = _silu(h)
    h = _bn(jnp.dot(h, cw2_ref[...], preferred_element_type=jnp.float32)
            + cb2_ref[...], cg2_ref[...], cbe2_ref[...])
    hchem_ref[...] = h
    proj_ref[...] = jnp.dot(x, sw1c_ref[...], preferred_element_type=jnp.float32)


def _chem_prep(chem_feats, cw1, cb1, cg1, cbe1, cw2, cb2, cg2, cbe2, sw1c, sb1):
    return pl.pallas_call(
        _chem_body,
        out_shape=(jax.ShapeDtypeStruct((NG, H), jnp.float32),
                   jax.ShapeDtypeStruct((NG, H), jnp.float32)),
    )(chem_feats, cw1, cb1, cg1, cbe1, cw2, cb2, cg2, cbe2, sw1c, sb1)


def _geom_body(geom_ref, verts_ref, npT_ref, gw1_ref, gb1_ref, gg1_ref,
               gbe1_ref, gw2_ref, gb2_ref, gg2_ref, gbe2_ref,
               hg_ref, d2_ref):
    x = geom_ref[...]
    h = _bn(jnp.dot(x, gw1_ref[...], preferred_element_type=jnp.float32)
            + gb1_ref[...], gg1_ref[...], gbe1_ref[...])
    h = _silu(h)
    h = _bn(jnp.dot(h, gw2_ref[...], preferred_element_type=jnp.float32)
            + gb2_ref[...], gg2_ref[...], gbe2_ref[...])
    hg_ref[...] = h
    v = verts_ref[...]
    npT = npT_ref[...]
    vsq = jnp.sum(v * v, axis=1, keepdims=True)
    nsq = jnp.sum(npT * npT, axis=0, keepdims=True)
    d2_ref[...] = vsq + nsq - 2.0 * jnp.dot(v, npT,
                                            preferred_element_type=jnp.float32)


def _geom_prep(geom_feats, verts, npT, gw1, gb1, gg1, gbe1, gw2, gb2, gg2, gbe2):
    B = 1000
    grid = NS // B
    return pl.pallas_call(
        _geom_body,
        grid=(grid,),
        in_specs=[
            pl.BlockSpec((B, DG), lambda i: (i, 0)),
            pl.BlockSpec((B, 3), lambda i: (i, 0)),
            pl.BlockSpec((3, NG), lambda i: (0, 0)),
            pl.BlockSpec((DG, H), lambda i: (0, 0)),
            pl.BlockSpec((1, H), lambda i: (0, 0)),
            pl.BlockSpec((1, H), lambda i: (0, 0)),
            pl.BlockSpec((1, H), lambda i: (0, 0)),
            pl.BlockSpec((H, H), lambda i: (0, 0)),
            pl.BlockSpec((1, H), lambda i: (0, 0)),
            pl.BlockSpec((1, H), lambda i: (0, 0)),
            pl.BlockSpec((1, H), lambda i: (0, 0)),
        ],
        out_specs=[
            pl.BlockSpec((B, H), lambda i: (i, 0)),
            pl.BlockSpec((B, NG), lambda i: (i, 0)),
        ],
        out_shape=(jax.ShapeDtypeStruct((NS, H), jnp.float32),
                   jax.ShapeDtypeStruct((NS, NG), jnp.float32)),
    )(geom_feats, verts, npT, gw1, gb1, gg1, gbe1, gw2, gb2, gg2, gbe2)


def _topk_body(d2_ref, idx_ref, scratch_ref):
    BT = d2_ref.shape[0]
    scratch_ref[...] = d2_ref[...]
    colid = jax.lax.broadcasted_iota(jnp.int32, (BT, NG), 1)
    kid = jax.lax.broadcasted_iota(jnp.int32, (BT, K), 1)

    def step(i, acc):
        d = scratch_ref[...]
        m = jnp.min(d, axis=1, keepdims=True)
        am = jnp.min(jnp.where(d == m, colid, NG), axis=1)
        scratch_ref[...] = jnp.where(colid == am[:, None], jnp.inf, d)
        return jnp.where(kid == i, am[:, None], acc)

    idx_ref[...] = jax.lax.fori_loop(0, K, step,
                                     jnp.zeros((BT, K), jnp.int32))


def _topk(d2):
    BT = 400
    grid = NS // BT
    return pl.pallas_call(
        _topk_body,
        grid=(grid,),
        in_specs=[pl.BlockSpec((BT, NG), lambda i: (i, 0))],
        out_specs=pl.BlockSpec((BT, K), lambda i: (i, 0)),
        out_shape=jax.ShapeDtypeStruct((NS, K), jnp.int32),
        scratch_shapes=[pltpu.VMEM((BT, NG), jnp.float32)],
    )(d2)


_GC = 128


def _sc_gather_body(tblc_ref, tbln_ref, idx_ref, chem_ref, npg_ref,
                    idx_v, chem_v, np_v, sem1, sem2):
    wid = lax.axis_index("s") * _SC_CORES + lax.axis_index("c")
    nchunk = E // _GC

    def do_chunk(ci):
        base = ci * _GC
        pltpu.sync_copy(idx_ref.at[pl.ds(base, _GC)], idx_v)
        cp1 = pltpu.async_copy(tblc_ref.at[idx_v], chem_v, sem1)
        cp2 = pltpu.async_copy(tbln_ref.at[idx_v], np_v, sem2)
        cp1.wait()
        cp2.wait()
        pltpu.sync_copy(chem_v, chem_ref.at[pl.ds(base, _GC)])
        pltpu.sync_copy(np_v, npg_ref.at[pl.ds(base, _GC)])

    def loop_body(j, carry):
        do_chunk(wid + j * _NW)
        return carry

    lax.fori_loop(0, nchunk // _NW, loop_body, 0)
    rem = nchunk - (nchunk // _NW) * _NW

    @pl.when(wid < rem)
    def _():
        do_chunk((nchunk // _NW) * _NW + wid)


def _sc_gather(tblc, tbln, idx_flat):
    return pl.kernel(
        _sc_gather_body,
        out_type=(jax.ShapeDtypeStruct((E, 64), jnp.int32),
                  jax.ShapeDtypeStruct((E, 16), jnp.float32)),
        mesh=plsc.VectorSubcoreMesh(core_axis_name="c", subcore_axis_name="s",
                                    num_cores=_SC_CORES,
                                    num_subcores=_SC_SUBCORES),
        scratch_types=[pltpu.VMEM((_GC,), jnp.int32),
                       pltpu.VMEM((_GC, 64), jnp.int32),
                       pltpu.VMEM((_GC, 16), jnp.float32),
                       pltpu.SemaphoreType.DMA,
                       pltpu.SemaphoreType.DMA],
        compiler_params=pltpu.CompilerParams(use_tc_tiling_on_sc=False),
    )(tblc, tbln, idx_flat)


def _edge_body(chem_g_ref, npg_ref, verts_ref, nrm_ref, hg1_ref, mu_ref,
               sw1_ref, sb1_ref, sg1_ref, sbe1_ref, sw2_ref, sb2_ref,
               sg2_ref, sbe2_ref, fw1_ref, fb1_ref, fg1_ref, fbe1_ref,
               fw2_ref, fb2_ref, fg2_ref, fbe2_ref, out_ref):
    EB = npg_ref.shape[0]
    BV = EB // K

    npg = npg_ref[...]
    dx = npg[:, 0:1] - verts_ref[:, 0:1]
    dy = npg[:, 1:2] - verts_ref[:, 1:2]
    dz = npg[:, 2:3] - verts_ref[:, 2:3]
    dist = jnp.sqrt(dx * dx + dy * dy + dz * dz)
    ang = (dx * nrm_ref[:, 0:1] + dy * nrm_ref[:, 1:2]
           + dz * nrm_ref[:, 2:3]) / dist

    mu_d = mu_ref[0:1, :]
    mu_a = mu_ref[1:2, :]
    enc_d = jnp.exp(-(((dist - mu_d) / 0.5) ** 2))
    enc_a = jnp.exp(-(((ang - mu_a) / 0.125) ** 2))

    enc = jnp.concatenate([enc_d, enc_a], axis=1)
    x1 = (jnp.dot(chem_g_ref[...], sw1_ref[:DC, :].astype(jnp.bfloat16),
                  preferred_element_type=jnp.float32)
          + jnp.dot(enc, sw1_ref[DC:, :], preferred_element_type=jnp.float32))
    h = _bn(x1 + sb1_ref[...], sg1_ref[...], sbe1_ref[...])
    h = _silu(h)
    h = _bn(jnp.dot(h.astype(jnp.bfloat16), sw2_ref[...].astype(jnp.bfloat16),
                    preferred_element_type=jnp.float32)
            + sb2_ref[...], sg2_ref[...], sbe2_ref[...])
    filt, core = h[:, :H], h[:, H:]
    he = jax.nn.sigmoid(filt) * jax.nn.softplus(core)
    h_cg = jnp.sum(he.reshape(BV, K, H), axis=1)

    y = jnp.concatenate([h_cg, hg1_ref[...]], axis=1)
    y = _bn(jnp.dot(y, fw1_ref[...], preferred_element_type=jnp.float32)
            + fb1_ref[...], fg1_ref[...], fbe1_ref[...])
    y = _silu(y)
    y = _bn(jnp.dot(y, fw2_ref[...], preferred_element_type=jnp.float32)
            + fb2_ref[...], fg2_ref[...], fbe2_ref[...])
    out_ref[...] = y


def _edge_final(chem_g, npg, verts, nrm, hg1, mu, sw1, sb1, sg1, sbe1, sw2,
                sb2, sg2, sbe2, fw1, fb1, fg1, fbe1, fw2, fb2, fg2, fbe2):
    BV = 400
    EB = BV * K
    grid = NS // BV
    const = lambda shape: pl.BlockSpec(shape, lambda i: (0, 0))
    return pl.pallas_call(
        _edge_body,
        grid=(grid,),
        in_specs=[
            pl.BlockSpec((EB, DC), lambda i: (i, 0)),
            pl.BlockSpec((EB, 16), lambda i: (i, 0)),
            pl.BlockSpec((EB, 3), lambda i: (i, 0)),
            pl.BlockSpec((EB, 3), lambda i: (i, 0)),
            pl.BlockSpec((BV, H), lambda i: (i, 0)),
            const((2, GDF)),
            const((DC + 2 * GDF, H)), const((1, H)), const((1, H)), const((1, H)),
            const((H, 2 * H)), const((1, 2 * H)), const((1, 2 * H)), const((1, 2 * H)),
            const((2 * H, H)), const((1, H)), const((1, H)), const((1, H)),
            const((H, H)), const((1, H)), const((1, H)), const((1, H)),
        ],
        out_specs=pl.BlockSpec((BV, H), lambda i: (i, 0)),
        out_shape=jax.ShapeDtypeStruct((NS, H), jnp.float32),
    )(chem_g, npg, verts, nrm, hg1, mu, sw1, sb1, sg1, sbe1, sw2, sb2, sg2,
      sbe2, fw1, fb1, fg1, fbe1, fw2, fb2, fg2, fbe2)


def kernel(chem_feats, geom_feats, verts, node_pos, cw1, cb1, cg1, cbe1, cw2,
           cb2, cg2, cbe2, gw1, gb1, gg1, gbe1, gw2, gb2, gg2, gbe2, sw1, sb1,
           sg1, sbe1, sw2, sb2, sg2, sbe2, fw1, fb1, fg1, fbe1, fw2, fb2, fg2,
           fbe2):
    row = lambda b: b.reshape(1, -1)
    h_chem, _ = _chem_prep(chem_feats, cw1, row(cb1), row(cg1), row(cbe1),
                           cw2, row(cb2), row(cg2), row(cbe2),
                           sw1[:DC], row(sb1))
    npT = node_pos.T
    hg1, d2 = _geom_prep(geom_feats, verts, npT, gw1, row(gb1), row(gg1),
                         row(gbe1), gw2, row(gb2), row(gg2), row(gbe2))
    idx = _topk(d2)
    flat = idx.reshape(-1)
    tblc = lax.bitcast_convert_type(
        chem_feats.astype(jnp.bfloat16).reshape(NG, 64, 2), jnp.int32)
    np16 = jnp.pad(node_pos, ((0, 0), (0, 13)))
    chem_i32, npg = _sc_gather(tblc, np16, flat)
    chem_g = lax.bitcast_convert_type(chem_i32, jnp.bfloat16).reshape(E, DC)
    verts_e = jnp.repeat(verts, K, axis=0)
    nrm_e = jnp.repeat(geom_feats[:, DG - 3:], K, axis=0)
    mu = jnp.stack([jnp.linspace(0.0, 8.0, GDF),
                    jnp.linspace(-1.0, 1.0, GDF)]).astype(jnp.float32)
    out = _edge_final(chem_g, npg, verts_e, nrm_e, hg1, mu, sw1, row(sb1), row(sg1),
                      row(sbe1), sw2, row(sb2), row(sg2), row(sbe2), fw1,
                      row(fb1), row(fg1), row(fbe1), fw2, row(fb2), row(fg2),
                      row(fbe2))
    return (out, h_chem)
